# baseline (speedup 1.0000x reference)
import jax
import jax.numpy as jnp
from jax.experimental import pallas as pl
from jax.experimental.pallas import tpu as pltpu


def _fused_cnn_kernel(x_ref, b1_ref, b1t_ref, b2_ref, b2t_ref, wf1_ref,
                      bf1_ref, wf2_ref, bf2_ref, o_ref):
    G = x_ref.shape[1]
    x = x_ref[...]

    lhs1 = jnp.concatenate([x[k:k + 24] for k in range(5)], axis=-1)
    lhs1 = lhs1.reshape(24 * G, 160)
    c0 = jnp.dot(lhs1, b1_ref[0], preferred_element_type=jnp.float32)
    c1 = jnp.dot(lhs1, b1_ref[1], preferred_element_type=jnp.float32)
    m1 = jnp.maximum(c0, c1).reshape(12, 2, G, 384)
    y1 = jnp.max(m1, axis=1)
    y1 = jnp.maximum(y1 + b1t_ref[...], 0.0)

    lhs2 = jnp.concatenate([y1[k:k + 8] for k in range(5)], axis=-1)
    lhs2 = lhs2.reshape(8 * G, 1920)
    d0 = jnp.dot(lhs2, b2_ref[0], preferred_element_type=jnp.float32)
    d1 = jnp.dot(lhs2, b2_ref[1], preferred_element_type=jnp.float32)
    m2 = jnp.maximum(d0, d1).reshape(4, 2, G, 256)
    y2 = jnp.max(m2, axis=1)
    y2 = jnp.maximum(y2 + b2t_ref[...], 0.0)

    h = jnp.dot(y2[0], wf1_ref[0:256], preferred_element_type=jnp.float32)
    h = h + jnp.dot(y2[1], wf1_ref[256:512],
                    preferred_element_type=jnp.float32)
    h = h + jnp.dot(y2[2], wf1_ref[512:768],
                    preferred_element_type=jnp.float32)
    h = h + jnp.dot(y2[3], wf1_ref[768:1024],
                    preferred_element_type=jnp.float32)
    h = jnp.maximum(h + bf1_ref[...], 0.0)

    logits = jnp.dot(h, wf2_ref[...],
                     preferred_element_type=jnp.float32) + bf2_ref[...]
    mx = jnp.max(logits, axis=1, keepdims=True)
    s = logits - mx
    lse = jnp.log(jnp.sum(jnp.exp(s), axis=1, keepdims=True))
    o_ref[...] = (s - lse)[:, :10].astype(o_ref.dtype)


def kernel(x, W1m, b1r, W2m, b2r, Wf1m, bf1r, Wf2m, bf2r):
    B = x.shape[0]
    xr = x.reshape(B, 28, 28)
    xt = jnp.transpose(xr.astype(jnp.bfloat16), (1, 0, 2))
    x32 = jnp.pad(xt, ((0, 0), (0, 0), (0, 4)))

    W1r = W1m[:25].reshape(5, 5, 32)
    w32 = jnp.arange(32)
    pw12 = jnp.arange(12)

    def build_b1(dj):
        kj = w32[:, None] - (2 * pw12[None, :] + dj)
        valid = (kj >= 0) & (kj < 5) & (w32[:, None] < 28)
        g = W1r[:, kj.clip(0, 4), :]
        return jnp.where(valid[None, :, :, None], g, 0.0).reshape(160, 384)

    b1 = jnp.stack([build_b1(0), build_b1(1)]).astype(jnp.bfloat16)
    b1t = jnp.tile(b1r, (1, 12))

    W2r = W2m.reshape(5, 5, 32, 64)

    def build_b2(dj):
        w_ar = jnp.arange(12)
        pw_ar = jnp.arange(4)
        kj = w_ar[:, None] - (2 * pw_ar[None, :] + dj)
        valid = (kj >= 0) & (kj < 5)
        g = W2r[:, kj.clip(0, 4), :, :]
        g = jnp.where(valid[None, :, :, None, None], g, 0.0)
        g = g.transpose(0, 1, 3, 2, 4)
        return g.reshape(1920, 256)

    b2 = jnp.stack([build_b2(0), build_b2(1)])
    b2t = jnp.tile(b2r, (1, 4)).reshape(1, 1, 256)

    G = 256 if B % 256 == 0 else 1
    return pl.pallas_call(
        _fused_cnn_kernel,
        out_shape=jax.ShapeDtypeStruct((B, 10), jnp.float32),
        grid=(B // G,),
        in_specs=[
            pl.BlockSpec((28, G, 32), lambda b: (0, b, 0)),
            pl.BlockSpec((2, 160, 384), lambda b: (0, 0, 0)),
            pl.BlockSpec((1, 384), lambda b: (0, 0)),
            pl.BlockSpec((2, 1920, 256), lambda b: (0, 0, 0)),
            pl.BlockSpec((1, 1, 256), lambda b: (0, 0, 0)),
            pl.BlockSpec((1024, 128), lambda b: (0, 0)),
            pl.BlockSpec((1, 128), lambda b: (0, 0)),
            pl.BlockSpec((128, 128), lambda b: (0, 0)),
            pl.BlockSpec((1, 128), lambda b: (0, 0)),
        ],
        out_specs=pl.BlockSpec((G, 10), lambda b: (b, 0)),
        compiler_params=pltpu.CompilerParams(
            dimension_semantics=("arbitrary",),
            vmem_limit_bytes=100 * 1024 * 1024,
        ),
    )(x32, b1, b1t, b2, b2t, Wf1m, bf1r, Wf2m, bf2r)


# speedup vs baseline: 1.0858x; 1.0858x over previous
import jax
import jax.numpy as jnp
from jax.experimental import pallas as pl
from jax.experimental.pallas import tpu as pltpu


def _fused_cnn_kernel(x_ref, w1_ref, b1r_ref, w2_ref, b2r_ref, wf1_ref,
                      bf1_ref, wf2_ref, bf2_ref, o_ref,
                      b1s, b1ts, b2s, b2ts):
    G = x_ref.shape[1]

    @pl.when(pl.program_id(0) == 0)
    def _build():
        b1s[...] = jnp.zeros_like(b1s)
        b2s[...] = jnp.zeros_like(b2s)
        for dj in range(2):
            for ki in range(5):
                for kj in range(5):
                    for pw in range(12):
                        r = ki * 32 + 2 * pw + dj + kj
                        b1s[dj, r:r + 1, pw * 32:(pw + 1) * 32] = (
                            w1_ref[ki * 5 + kj:ki * 5 + kj + 1, :])
                    for pw in range(4):
                        r = ki * 384 + (2 * pw + dj + kj) * 32
                        s = (ki * 5 + kj) * 32
                        b2s[dj, r:r + 32, pw * 64:(pw + 1) * 64] = (
                            w2_ref[s:s + 32, :])
        for pw in range(12):
            b1ts[0:1, pw * 32:(pw + 1) * 32] = b1r_ref[...]
        for pw in range(4):
            b2ts[0:1, pw * 64:(pw + 1) * 64] = b2r_ref[...]

    x = x_ref[...]

    lhs1 = jnp.concatenate([x[k:k + 24] for k in range(5)], axis=-1)
    lhs1 = lhs1.reshape(24 * G, 160)
    c0 = jnp.dot(lhs1, b1s[0], preferred_element_type=jnp.float32)
    c1 = jnp.dot(lhs1, b1s[1], preferred_element_type=jnp.float32)
    m1 = jnp.maximum(c0, c1).reshape(12, 2, G, 384)
    y1 = jnp.max(m1, axis=1)
    y1 = jnp.maximum(y1 + b1ts[...], 0.0)

    lhs2 = jnp.concatenate([y1[k:k + 8] for k in range(5)], axis=-1)
    lhs2 = lhs2.reshape(8 * G, 1920)
    d0 = jnp.dot(lhs2, b2s[0], preferred_element_type=jnp.float32)
    d1 = jnp.dot(lhs2, b2s[1], preferred_element_type=jnp.float32)
    m2 = jnp.maximum(d0, d1).reshape(4, 2, G, 256)
    y2 = jnp.max(m2, axis=1)
    y2 = jnp.maximum(y2 + b2ts[...], 0.0)

    h = jnp.dot(y2[0], wf1_ref[0:256], preferred_element_type=jnp.float32)
    h = h + jnp.dot(y2[1], wf1_ref[256:512],
                    preferred_element_type=jnp.float32)
    h = h + jnp.dot(y2[2], wf1_ref[512:768],
                    preferred_element_type=jnp.float32)
    h = h + jnp.dot(y2[3], wf1_ref[768:1024],
                    preferred_element_type=jnp.float32)
    h = jnp.maximum(h + bf1_ref[...], 0.0)

    logits = jnp.dot(h, wf2_ref[...],
                     preferred_element_type=jnp.float32) + bf2_ref[...]
    mx = jnp.max(logits, axis=1, keepdims=True)
    s = logits - mx
    lse = jnp.log(jnp.sum(jnp.exp(s), axis=1, keepdims=True))
    o_ref[...] = (s - lse)[:, :10].astype(o_ref.dtype)


def kernel(x, W1m, b1r, W2m, b2r, Wf1m, bf1r, Wf2m, bf2r):
    B = x.shape[0]
    xr = x.reshape(B, 28, 28)
    xt = jnp.transpose(xr, (1, 0, 2))
    x32 = jnp.pad(xt, ((0, 0), (0, 0), (0, 4)))

    G = 256 if B % 256 == 0 else 1
    return pl.pallas_call(
        _fused_cnn_kernel,
        out_shape=jax.ShapeDtypeStruct((B, 10), jnp.float32),
        grid=(B // G,),
        in_specs=[
            pl.BlockSpec((28, G, 32), lambda b: (0, b, 0)),
            pl.BlockSpec((32, 32), lambda b: (0, 0)),
            pl.BlockSpec((1, 32), lambda b: (0, 0)),
            pl.BlockSpec((800, 64), lambda b: (0, 0)),
            pl.BlockSpec((1, 64), lambda b: (0, 0)),
            pl.BlockSpec((1024, 128), lambda b: (0, 0)),
            pl.BlockSpec((1, 128), lambda b: (0, 0)),
            pl.BlockSpec((128, 128), lambda b: (0, 0)),
            pl.BlockSpec((1, 128), lambda b: (0, 0)),
        ],
        out_specs=pl.BlockSpec((G, 10), lambda b: (b, 0)),
        scratch_shapes=[
            pltpu.VMEM((2, 160, 384), jnp.float32),
            pltpu.VMEM((1, 384), jnp.float32),
            pltpu.VMEM((2, 1920, 256), jnp.float32),
            pltpu.VMEM((1, 256), jnp.float32),
        ],
        compiler_params=pltpu.CompilerParams(
            dimension_semantics=("arbitrary",),
            vmem_limit_bytes=100 * 1024 * 1024,
        ),
    )(x32, W1m, b1r, W2m, b2r, Wf1m, bf1r, Wf2m, bf2r)
